# baseline (speedup 1.0000x reference)
import functools

import jax
import jax.numpy as jnp
from jax import lax
from jax.experimental import pallas as pl
from jax.experimental.pallas import tpu as pltpu
from jax.experimental.pallas import tpu_sc as plsc

_N = 10000
_E = 320000
_HID = 128
_BE = 6400
_X = _BE // 32
_NB = _E // _BE
_ROWS = _E // 32


def _edge_rel_sc(xs, ys, zs, src, dst):
    info = plsc.get_sparse_core_info()
    nc, ns = info.num_cores, info.num_subcores
    nw = nc * ns
    wrows = _ROWS // nw + 1
    sspan = 3 * _BE

    mesh = plsc.VectorSubcoreMesh(core_axis_name="c", subcore_axis_name="s")

    @functools.partial(
        pl.kernel,
        out_type=jax.ShapeDtypeStruct((_ROWS * 128,), jnp.float32),
        mesh=mesh,
        scratch_types=[
            pltpu.VMEM((_N,), jnp.float32),
            pltpu.VMEM((_N,), jnp.float32),
            pltpu.VMEM((_N,), jnp.float32),
            pltpu.VMEM((sspan,), jnp.int32),
            pltpu.VMEM((sspan,), jnp.int32),
            pltpu.VMEM((wrows * 128,), jnp.float32),
        ],
        compiler_params=pltpu.CompilerParams(needs_layout_passes=False),
    )
    def sc_gather(xs_h, ys_h, zs_h, src_h, dst_h, out_h,
                  xs_v, ys_v, zs_v, src_v, dst_v, out_v):
        wid = lax.axis_index("s") * nc + lax.axis_index("c")
        r0 = (_ROWS * wid) >> 5
        i0 = (r0 * 5243) >> 20
        x0 = r0 - 200 * i0
        sblk = jnp.minimum(i0, _NB - 3)
        pltpu.sync_copy(xs_h, xs_v)
        pltpu.sync_copy(ys_h, ys_v)
        pltpu.sync_copy(zs_h, zs_v)
        pltpu.sync_copy(src_h.at[pl.ds(sblk * _BE, sspan)], src_v)
        pltpu.sync_copy(dst_h.at[pl.ds(sblk * _BE, sspan)], dst_v)
        lanes4 = lax.iota(jnp.int32, 16) * 4
        iota200 = lax.iota(jnp.int32, 16) * 200
        iblk0 = i0 - sblk

        @plsc.parallel_loop(0, 2 * wrows, unroll=8)
        def body(g):
            r_off = g >> 1
            jg = g & 1
            xm = x0 + r_off
            i_add = ((xm >= 200).astype(jnp.int32)
                     + (xm >= 400).astype(jnp.int32))
            x = xm - 200 * i_add
            ebase = (iblk0 + i_add) * _BE + jg * 3200 + x
            idxv = iota200 + ebase
            si = plsc.load_gather(src_v, [idxv])
            di = plsc.load_gather(dst_v, [idxv])
            obase = r_off * 128 + jg * 64
            d2 = None
            for c, tab in ((0, xs_v), (1, ys_v), (2, zs_v)):
                a = plsc.load_gather(tab, [si])
                b = plsc.load_gather(tab, [di])
                r = a - b
                d2 = r * r if d2 is None else d2 + r * r
                plsc.store_scatter(out_v, [lanes4 + (obase + c)], r)
            plsc.store_scatter(out_v, [lanes4 + (obase + 3)], d2)

        pltpu.sync_copy(out_v, out_h.at[pl.ds(r0 * 128, wrows * 128)])

    return sc_gather(xs, ys, zs, src, dst)


def _edge_mlp_body(f_ref, w0_ref, b0_ref, w1_ref, b1_ref, w2_ref, b2_ref,
                   g_ref, be_ref, out_ref):
    f = f_ref[...]
    m3 = (lax.broadcasted_iota(jnp.int32, f.shape, 1) & 3) == 3
    f = jnp.where(m3, jnp.sqrt(jnp.abs(f)), f)
    H = jnp.dot(f, w0_ref[...], preferred_element_type=jnp.float32)
    w1 = w1_ref[...]
    w2 = w2_ref[...]
    b1 = b1_ref[...]
    b2 = b2_ref[...]
    gg = g_ref[...]
    bb = be_ref[...]
    b0 = b0_ref[...]
    for j in range(32):
        h = jnp.maximum(H[:, 128 * j:128 * (j + 1)] + b0, 0.0)
        h = jnp.maximum(
            jnp.dot(h, w1, preferred_element_type=jnp.float32) + b1, 0.0)
        h = jnp.dot(h, w2, preferred_element_type=jnp.float32) + b2
        mu = jnp.mean(h, axis=1, keepdims=True)
        d = h - mu
        var = jnp.mean(d * d, axis=1, keepdims=True)
        out_ref[j] = d * (lax.rsqrt(var + 1e-5) * gg) + bb


def _edge_mlp(feat, weights, interpret=False):
    wspec = lambda shp: pl.BlockSpec(shp, lambda i: tuple(0 for _ in shp))
    return pl.pallas_call(
        _edge_mlp_body,
        grid=(_NB,),
        in_specs=[
            pl.BlockSpec((_X, 128), lambda i: (i, 0)),
            wspec((_HID, 4096)),
            wspec((1, _HID)),
            wspec((_HID, _HID)),
            wspec((1, _HID)),
            wspec((_HID, _HID)),
            wspec((1, _HID)),
            wspec((1, _HID)),
            wspec((1, _HID)),
        ],
        out_specs=pl.BlockSpec((32, _X, 128), lambda i: (i, 0, 0)),
        out_shape=jax.ShapeDtypeStruct((_NB * 32, _X, 128), jnp.float32),
        compiler_params=pltpu.CompilerParams(
            dimension_semantics=("arbitrary",)),
        interpret=interpret,
    )(feat, *weights)


def _node_mlp_body(vel_ref, mat_ref, w0_ref, b0_ref, w1_ref, b1_ref, w2_ref,
                   b2_ref, g_ref, be_ref, out_ref):
    bn = vel_ref.shape[0]
    onehot = (mat_ref[...] == lax.broadcasted_iota(jnp.int32, (bn, 16), 1)
              ).astype(jnp.float32)
    lhs = jnp.concatenate([vel_ref[...], onehot], axis=1)
    h = jnp.dot(lhs, w0_ref[...], preferred_element_type=jnp.float32) + b0_ref[...]
    h = jnp.maximum(h, 0.0)
    h = jnp.dot(h, w1_ref[...], preferred_element_type=jnp.float32) + b1_ref[...]
    h = jnp.maximum(h, 0.0)
    h = jnp.dot(h, w2_ref[...], preferred_element_type=jnp.float32) + b2_ref[...]
    mu = jnp.mean(h, axis=1, keepdims=True)
    d = h - mu
    var = jnp.mean(d * d, axis=1, keepdims=True)
    out_ref[...] = d * (lax.rsqrt(var + 1e-5) * g_ref[...]) + be_ref[...]


def _node_mlp(vel16, mats, W0cat, b0p, nW1, nb1, nW2, nb2, n_gamma, n_beta,
              interpret=False):
    BN = 2000
    grid = _N // BN
    wspec = lambda shp: pl.BlockSpec(shp, lambda i: (0, 0))
    return pl.pallas_call(
        _node_mlp_body,
        grid=(grid,),
        in_specs=[
            pl.BlockSpec((BN, 16), lambda i: (i, 0)),
            pl.BlockSpec((BN, 1), lambda i: (i, 0)),
            wspec((32, _HID)),
            wspec((1, _HID)),
            wspec((_HID, _HID)),
            wspec((1, _HID)),
            wspec((_HID, _HID)),
            wspec((1, _HID)),
            wspec((1, _HID)),
            wspec((1, _HID)),
        ],
        out_specs=pl.BlockSpec((BN, _HID), lambda i: (i, 0)),
        out_shape=jax.ShapeDtypeStruct((_N, _HID), jnp.float32),
        compiler_params=pltpu.CompilerParams(
            dimension_semantics=("arbitrary",)),
        interpret=interpret,
    )(vel16, mats, W0cat, b0p.reshape(1, -1), nW1, nb1.reshape(1, -1), nW2,
      nb2.reshape(1, -1), n_gamma.reshape(1, -1), n_beta.reshape(1, -1))


def kernel(materials, velocities, positions, neighbor_idxs, mat_W, mat_b,
           nW0, nb0, nW1, nb1, nW2, nb2, n_gamma, n_beta,
           eW0, eb0, eW1, eb1, eW2, eb2, e_gamma, e_beta):
    xs = positions[:, 0]
    ys = positions[:, 1]
    zs = positions[:, 2]
    src = neighbor_idxs[:, 0].astype(jnp.int32)
    dst = neighbor_idxs[:, 1].astype(jnp.int32)
    rel_flat = _edge_rel_sc(xs, ys, zs, src, dst)
    feat = rel_flat.reshape(_ROWS, 128)
    weights = [jnp.kron(jnp.eye(32, dtype=jnp.float32), eW0),
               eb0.reshape(1, -1), eW1, eb1.reshape(1, -1), eW2,
               eb2.reshape(1, -1), e_gamma.reshape(1, -1),
               e_beta.reshape(1, -1)]
    edges = _edge_mlp(feat, weights).reshape(_E, _HID)

    vel16 = jnp.pad(velocities.reshape(_N, 15), ((0, 0), (0, 1)))
    mats = materials.reshape(_N, 1).astype(jnp.int32)
    W0cat = jnp.concatenate(
        [nW0[:15], jnp.zeros((1, _HID), jnp.float32), mat_W @ nW0[15:],
         jnp.zeros((7, _HID), jnp.float32)], axis=0)
    b0p = nb0 + mat_b @ nW0[15:]
    nodes = _node_mlp(vel16, mats, W0cat, b0p, nW1, nb1, nW2, nb2,
                      n_gamma, n_beta)

    return (nodes, edges, neighbor_idxs)


# speedup vs baseline: 1.0055x; 1.0055x over previous
import functools

import jax
import jax.numpy as jnp
from jax import lax
from jax.experimental import pallas as pl
from jax.experimental.pallas import tpu as pltpu
from jax.experimental.pallas import tpu_sc as plsc

_N = 10000
_E = 320000
_HID = 128
_BE = 12800
_X = _BE // 32
_NB = _E // _BE
_ROWS = _E // 32


def _edge_rel_sc(xs, ys, zs, src, dst):
    info = plsc.get_sparse_core_info()
    nc, ns = info.num_cores, info.num_subcores
    nw = nc * ns
    wrows = _ROWS // nw + 1
    sspan = 2 * _BE

    mesh = plsc.VectorSubcoreMesh(core_axis_name="c", subcore_axis_name="s")

    @functools.partial(
        pl.kernel,
        out_type=jax.ShapeDtypeStruct((_ROWS * 128,), jnp.float32),
        mesh=mesh,
        scratch_types=[
            pltpu.VMEM((_N,), jnp.float32),
            pltpu.VMEM((_N,), jnp.float32),
            pltpu.VMEM((_N,), jnp.float32),
            pltpu.VMEM((sspan,), jnp.int32),
            pltpu.VMEM((sspan,), jnp.int32),
            pltpu.VMEM((wrows * 128,), jnp.float32),
        ],
        compiler_params=pltpu.CompilerParams(needs_layout_passes=False),
    )
    def sc_gather(xs_h, ys_h, zs_h, src_h, dst_h, out_h,
                  xs_v, ys_v, zs_v, src_v, dst_v, out_v):
        wid = lax.axis_index("s") * nc + lax.axis_index("c")
        r0 = (_ROWS * wid) >> 5
        i0 = (r0 * 5243) >> 21
        x0 = r0 - 400 * i0
        sblk = jnp.minimum(i0, _NB - 2)
        pltpu.sync_copy(xs_h, xs_v)
        pltpu.sync_copy(ys_h, ys_v)
        pltpu.sync_copy(zs_h, zs_v)
        pltpu.sync_copy(src_h.at[pl.ds(sblk * _BE, sspan)], src_v)
        pltpu.sync_copy(dst_h.at[pl.ds(sblk * _BE, sspan)], dst_v)
        lanes4 = lax.iota(jnp.int32, 16) * 4
        iota400 = lax.iota(jnp.int32, 16) * 400
        iblk0 = i0 - sblk

        @plsc.parallel_loop(0, 2 * wrows, unroll=8)
        def body(g):
            r_off = g >> 1
            jg = g & 1
            xm = x0 + r_off
            i_add = (xm >= 400).astype(jnp.int32)
            x = xm - 400 * i_add
            ebase = (iblk0 + i_add) * _BE + jg * 6400 + x
            idxv = iota400 + ebase
            si = plsc.load_gather(src_v, [idxv])
            di = plsc.load_gather(dst_v, [idxv])
            obase = r_off * 128 + jg * 64
            d2 = None
            for c, tab in ((0, xs_v), (1, ys_v), (2, zs_v)):
                a = plsc.load_gather(tab, [si])
                b = plsc.load_gather(tab, [di])
                r = a - b
                d2 = r * r if d2 is None else d2 + r * r
                plsc.store_scatter(out_v, [lanes4 + (obase + c)], r)
            plsc.store_scatter(out_v, [lanes4 + (obase + 3)], d2)

        pltpu.sync_copy(out_v, out_h.at[pl.ds(r0 * 128, wrows * 128)])

    return sc_gather(xs, ys, zs, src, dst)


def _edge_mlp_body(f_ref, w0_ref, b0_ref, w1_ref, b1_ref, w2_ref, b2_ref,
                   g_ref, be_ref, out_ref):
    f = f_ref[...]
    m3 = (lax.broadcasted_iota(jnp.int32, f.shape, 1) & 3) == 3
    f = jnp.where(m3, jnp.sqrt(jnp.abs(f)), f)
    H = jnp.dot(f, w0_ref[...], preferred_element_type=jnp.float32)
    w1 = w1_ref[...]
    w2 = w2_ref[...]
    b1 = b1_ref[...]
    b2 = b2_ref[...]
    gg = g_ref[...]
    bb = be_ref[...]
    b0 = b0_ref[...]
    for j in range(32):
        h = jnp.maximum(H[:, 128 * j:128 * (j + 1)] + b0, 0.0)
        h = jnp.maximum(
            jnp.dot(h, w1, preferred_element_type=jnp.float32) + b1, 0.0)
        h = jnp.dot(h, w2, preferred_element_type=jnp.float32) + b2
        mu = jnp.mean(h, axis=1, keepdims=True)
        d = h - mu
        var = jnp.mean(d * d, axis=1, keepdims=True)
        out_ref[j] = d * (lax.rsqrt(var + 1e-5) * gg) + bb


def _edge_mlp(feat, weights, interpret=False):
    wspec = lambda shp: pl.BlockSpec(shp, lambda i: tuple(0 for _ in shp))
    return pl.pallas_call(
        _edge_mlp_body,
        grid=(_NB,),
        in_specs=[
            pl.BlockSpec((_X, 128), lambda i: (i, 0)),
            wspec((_HID, 4096)),
            wspec((1, _HID)),
            wspec((_HID, _HID)),
            wspec((1, _HID)),
            wspec((_HID, _HID)),
            wspec((1, _HID)),
            wspec((1, _HID)),
            wspec((1, _HID)),
        ],
        out_specs=pl.BlockSpec((32, _X, 128), lambda i: (i, 0, 0)),
        out_shape=jax.ShapeDtypeStruct((_NB * 32, _X, 128), jnp.float32),
        compiler_params=pltpu.CompilerParams(
            dimension_semantics=("arbitrary",)),
        interpret=interpret,
    )(feat, *weights)


def _node_mlp_body(vel_ref, mat_ref, w0_ref, b0_ref, w1_ref, b1_ref, w2_ref,
                   b2_ref, g_ref, be_ref, out_ref):
    bn = vel_ref.shape[0]
    onehot = (mat_ref[...] == lax.broadcasted_iota(jnp.int32, (bn, 16), 1)
              ).astype(jnp.float32)
    lhs = jnp.concatenate([vel_ref[...], onehot], axis=1)
    h = jnp.dot(lhs, w0_ref[...], preferred_element_type=jnp.float32) + b0_ref[...]
    h = jnp.maximum(h, 0.0)
    h = jnp.dot(h, w1_ref[...], preferred_element_type=jnp.float32) + b1_ref[...]
    h = jnp.maximum(h, 0.0)
    h = jnp.dot(h, w2_ref[...], preferred_element_type=jnp.float32) + b2_ref[...]
    mu = jnp.mean(h, axis=1, keepdims=True)
    d = h - mu
    var = jnp.mean(d * d, axis=1, keepdims=True)
    out_ref[...] = d * (lax.rsqrt(var + 1e-5) * g_ref[...]) + be_ref[...]


def _node_mlp(vel16, mats, W0cat, b0p, nW1, nb1, nW2, nb2, n_gamma, n_beta,
              interpret=False):
    BN = 2000
    grid = _N // BN
    wspec = lambda shp: pl.BlockSpec(shp, lambda i: (0, 0))
    return pl.pallas_call(
        _node_mlp_body,
        grid=(grid,),
        in_specs=[
            pl.BlockSpec((BN, 16), lambda i: (i, 0)),
            pl.BlockSpec((BN, 1), lambda i: (i, 0)),
            wspec((32, _HID)),
            wspec((1, _HID)),
            wspec((_HID, _HID)),
            wspec((1, _HID)),
            wspec((_HID, _HID)),
            wspec((1, _HID)),
            wspec((1, _HID)),
            wspec((1, _HID)),
        ],
        out_specs=pl.BlockSpec((BN, _HID), lambda i: (i, 0)),
        out_shape=jax.ShapeDtypeStruct((_N, _HID), jnp.float32),
        compiler_params=pltpu.CompilerParams(
            dimension_semantics=("arbitrary",)),
        interpret=interpret,
    )(vel16, mats, W0cat, b0p.reshape(1, -1), nW1, nb1.reshape(1, -1), nW2,
      nb2.reshape(1, -1), n_gamma.reshape(1, -1), n_beta.reshape(1, -1))


def kernel(materials, velocities, positions, neighbor_idxs, mat_W, mat_b,
           nW0, nb0, nW1, nb1, nW2, nb2, n_gamma, n_beta,
           eW0, eb0, eW1, eb1, eW2, eb2, e_gamma, e_beta):
    xs = positions[:, 0]
    ys = positions[:, 1]
    zs = positions[:, 2]
    src = neighbor_idxs[:, 0].astype(jnp.int32)
    dst = neighbor_idxs[:, 1].astype(jnp.int32)
    rel_flat = _edge_rel_sc(xs, ys, zs, src, dst)
    feat = rel_flat.reshape(_ROWS, 128)
    weights = [jnp.kron(jnp.eye(32, dtype=jnp.float32), eW0),
               eb0.reshape(1, -1), eW1, eb1.reshape(1, -1), eW2,
               eb2.reshape(1, -1), e_gamma.reshape(1, -1),
               e_beta.reshape(1, -1)]
    edges = _edge_mlp(feat, weights).reshape(_E, _HID)

    vel16 = jnp.pad(velocities.reshape(_N, 15), ((0, 0), (0, 1)))
    mats = materials.reshape(_N, 1).astype(jnp.int32)
    W0cat = jnp.concatenate(
        [nW0[:15], jnp.zeros((1, _HID), jnp.float32), mat_W @ nW0[15:],
         jnp.zeros((7, _HID), jnp.float32)], axis=0)
    b0p = nb0 + mat_b @ nW0[15:]
    nodes = _node_mlp(vel16, mats, W0cat, b0p, nW1, nb1, nW2, nb2,
                      n_gamma, n_beta)

    return (nodes, edges, neighbor_idxs)
